# initial kernel scaffold (unmeasured)
import jax
import jax.numpy as jnp
from jax import lax
from jax.experimental import pallas as pl
from jax.experimental.pallas import tpu as pltpu

N_DEV = 4

_CompilerParams = getattr(pltpu, "CompilerParams", None) or getattr(
    pltpu, "TPUCompilerParams"
)
_ANY = getattr(pltpu, "ANY", None) or pltpu.MemorySpace.ANY


def _cast_bf16(x):
    m, k = x.shape
    blk = m // 8

    def body(x_ref, o_ref):
        o_ref[...] = x_ref[...].astype(jnp.bfloat16)

    return pl.pallas_call(
        body,
        grid=(8,),
        in_specs=[pl.BlockSpec((blk, k), lambda i: (i, 0))],
        out_specs=pl.BlockSpec((blk, k), lambda i: (i, 0)),
        out_shape=jax.ShapeDtypeStruct((m, k), jnp.bfloat16),
    )(x)


def kernel(x, w_mat):
    m_glob, k_shard = x.shape
    k_glob, n_glob = w_mat.shape
    m_per = m_glob // N_DEV
    k_blk = k_glob // N_DEV
    n_blk = n_glob // 2

    x_bf = _cast_bf16(x)

    my = lax.axis_index("i")
    s = (my + jnp.arange(N_DEV, dtype=jnp.int32)) % N_DEV

    def body(s_ref, x_ref, w_ref, o_ref, comm_ref, send_sems, recv_sems,
             loc_sem):
        n = pl.program_id(0)
        k = pl.program_id(1)
        me = s_ref[0]
        jj = s_ref[k]

        @pl.when((n == 0) & (k == 0))
        def _():
            barrier_sem = pltpu.get_barrier_semaphore()
            for t in range(1, N_DEV):
                pl.semaphore_signal(
                    barrier_sem, inc=1,
                    device_id=(s_ref[t],),
                    device_id_type=pl.DeviceIdType.MESH,
                )
            pl.semaphore_wait(barrier_sem, N_DEV - 1)

            loc = pltpu.make_async_copy(
                x_ref.at[pl.ds(me * m_per, m_per), :], comm_ref.at[me],
                loc_sem,
            )
            loc.start()

            for t in (3, 2, 1):
                tgt = s_ref[t]
                pltpu.make_async_remote_copy(
                    src_ref=x_ref.at[pl.ds(tgt * m_per, m_per), :],
                    dst_ref=comm_ref.at[me],
                    send_sem=send_sems.at[t],
                    recv_sem=recv_sems.at[me],
                    device_id=(tgt,),
                    device_id_type=pl.DeviceIdType.MESH,
                ).start()
            loc.wait()

        @pl.when((n == 0) & (k > 0))
        def _():
            pltpu.make_async_remote_copy(
                src_ref=comm_ref.at[jj],
                dst_ref=comm_ref.at[jj],
                send_sem=send_sems.at[0],
                recv_sem=recv_sems.at[jj],
                device_id=(me,),
                device_id_type=pl.DeviceIdType.MESH,
            ).wait_recv()

        acc = jnp.dot(
            comm_ref[jj],
            w_ref[...].astype(jnp.bfloat16),
            preferred_element_type=jnp.float32,
        )

        @pl.when(k == 0)
        def _():
            o_ref[...] = acc

        @pl.when(k > 0)
        def _():
            o_ref[...] += acc

        @pl.when(k == N_DEV - 1)
        def _():
            y = o_ref[...]
            o_ref[...] = y * jax.nn.sigmoid(y)

        @pl.when((n == 1) & (k == N_DEV - 1))
        def _():
            for t in (1, 2, 3):
                pltpu.make_async_remote_copy(
                    src_ref=x_ref.at[pl.ds(s_ref[t] * m_per, m_per), :],
                    dst_ref=comm_ref.at[me],
                    send_sem=send_sems.at[t],
                    recv_sem=recv_sems.at[me],
                    device_id=(s_ref[t],),
                    device_id_type=pl.DeviceIdType.MESH,
                ).wait_send()

    grid_spec = pltpu.PrefetchScalarGridSpec(
        num_scalar_prefetch=1,
        grid=(2, N_DEV),
        in_specs=[
            pl.BlockSpec(memory_space=_ANY),
            pl.BlockSpec((k_blk, n_blk), lambda n, k, s: (s[k], n)),
        ],
        out_specs=pl.BlockSpec((m_per, n_blk), lambda n, k, s: (0, n)),
        scratch_shapes=[
            pltpu.VMEM((N_DEV, m_per, k_shard), jnp.bfloat16),
            pltpu.SemaphoreType.DMA((N_DEV,)),
            pltpu.SemaphoreType.DMA((N_DEV,)),
            pltpu.SemaphoreType.DMA,
        ],
    )

    return pl.pallas_call(
        body,
        grid_spec=grid_spec,
        out_shape=jax.ShapeDtypeStruct((m_per, n_glob), jnp.float32),
        compiler_params=_CompilerParams(
            collective_id=0,
            dimension_semantics=("arbitrary", "arbitrary"),
        ),
    )(s, x_bf, w_mat)


# baseline (device time: 589497 ns/iter reference)
import jax
import jax.numpy as jnp
from jax import lax
from jax.experimental import pallas as pl
from jax.experimental.pallas import tpu as pltpu

N_DEV = 4
N_TILES = 4

_CompilerParams = getattr(pltpu, "CompilerParams", None) or getattr(
    pltpu, "TPUCompilerParams"
)


def _cast_bf16(x):
    m, k = x.shape
    blk = m // 8

    def body(x_ref, o_ref):
        o_ref[...] = x_ref[...].astype(jnp.bfloat16)

    return pl.pallas_call(
        body,
        grid=(8,),
        in_specs=[pl.BlockSpec((blk, k), lambda i: (i, 0))],
        out_specs=pl.BlockSpec((blk, k), lambda i: (i, 0)),
        out_shape=jax.ShapeDtypeStruct((m, k), jnp.bfloat16),
    )(x)


def kernel(x, w_mat):
    m_glob, k_shard = x.shape
    k_glob, n_glob = w_mat.shape
    m_per = m_glob // N_DEV
    k_blk = k_glob // N_DEV
    n_blk = n_glob // N_TILES

    x_bf = _cast_bf16(x)

    my = lax.axis_index("i")
    s = (my + jnp.arange(N_DEV, dtype=jnp.int32)) % N_DEV

    def body(s_ref, x_ref, w_ref, o_ref, comm_ref, xt_ref, send_sems,
             recv_sems, loc_sem):
        n = pl.program_id(0)
        k = pl.program_id(1)
        me = s_ref[0]
        jj = s_ref[k]

        @pl.when((n == 0) & (k == 0))
        def _():
            barrier_sem = pltpu.get_barrier_semaphore()
            for t in range(1, N_DEV):
                pl.semaphore_signal(
                    barrier_sem, inc=1,
                    device_id=(s_ref[t],),
                    device_id_type=pl.DeviceIdType.MESH,
                )
            pl.semaphore_wait(barrier_sem, N_DEV - 1)

            loc = pltpu.make_async_copy(
                x_ref.at[pl.ds(me * m_per, m_per), :], comm_ref.at[me],
                loc_sem,
            )
            loc.start()

            for t in (3, 2, 1):
                tgt = s_ref[t]
                pltpu.make_async_remote_copy(
                    src_ref=x_ref.at[pl.ds(tgt * m_per, m_per), :],
                    dst_ref=comm_ref.at[me],
                    send_sem=send_sems.at[t],
                    recv_sem=recv_sems.at[me],
                    device_id=(tgt,),
                    device_id_type=pl.DeviceIdType.MESH,
                ).start()
            loc.wait()

        @pl.when((n == 0) & (k > 0))
        def _():
            pltpu.make_async_remote_copy(
                src_ref=comm_ref.at[jj],
                dst_ref=comm_ref.at[jj],
                send_sem=send_sems.at[0],
                recv_sem=recv_sems.at[jj],
                device_id=(me,),
                device_id_type=pl.DeviceIdType.MESH,
            ).wait_recv()

        cp = pltpu.make_async_copy(comm_ref.at[jj], xt_ref, loc_sem)
        cp.start()
        cp.wait()

        acc = jnp.dot(
            xt_ref[...],
            w_ref[...].astype(jnp.bfloat16),
            preferred_element_type=jnp.float32,
        )

        @pl.when(k == 0)
        def _():
            o_ref[...] = acc

        @pl.when(k > 0)
        def _():
            o_ref[...] += acc

        @pl.when(k == N_DEV - 1)
        def _():
            y = o_ref[...]
            o_ref[...] = y * jax.nn.sigmoid(y)

        @pl.when((n == N_TILES - 1) & (k == N_DEV - 1))
        def _():
            for t in (1, 2, 3):
                pltpu.make_async_remote_copy(
                    src_ref=x_ref.at[pl.ds(s_ref[t] * m_per, m_per), :],
                    dst_ref=comm_ref.at[me],
                    send_sem=send_sems.at[t],
                    recv_sem=recv_sems.at[me],
                    device_id=(s_ref[t],),
                    device_id_type=pl.DeviceIdType.MESH,
                ).wait_send()

    grid_spec = pltpu.PrefetchScalarGridSpec(
        num_scalar_prefetch=1,
        grid=(N_TILES, N_DEV),
        in_specs=[
            pl.BlockSpec(memory_space=pl.ANY),
            pl.BlockSpec((k_blk, n_blk), lambda n, k, s: (s[k], n)),
        ],
        out_specs=[
            pl.BlockSpec((m_per, n_blk), lambda n, k, s: (0, n)),
            pl.BlockSpec(memory_space=pl.ANY),
        ],
        scratch_shapes=[
            pltpu.VMEM((m_per, k_blk), jnp.bfloat16),
            pltpu.SemaphoreType.DMA((N_DEV,)),
            pltpu.SemaphoreType.DMA((N_DEV,)),
            pltpu.SemaphoreType.DMA,
        ],
    )

    out, _ = pl.pallas_call(
        body,
        grid_spec=grid_spec,
        out_shape=[
            jax.ShapeDtypeStruct((m_per, n_glob), jnp.float32),
            jax.ShapeDtypeStruct((N_DEV, m_per, k_blk), jnp.bfloat16),
        ],
        compiler_params=_CompilerParams(
            collective_id=0,
            dimension_semantics=("arbitrary", "arbitrary"),
            vmem_limit_bytes=60 * 1024 * 1024,
        ),
    )(s, x_bf, w_mat)
    return out
